# baseline (speedup 1.0000x reference)
import functools

import jax
import jax.numpy as jnp
from jax import lax
from jax.experimental import pallas as pl
from jax.experimental.pallas import tpu as pltpu
from jax.experimental.pallas import tpu_sc as plsc

NC = 2
NS = 16
L = 16
NW = NC * NS

B = 4096
D = 64
B_PER_W = B // NW


def _mf_kernel(user_idx_hbm, item_idx_hbm, user_table_hbm, item_table_hbm,
               out_hbm, idx_u_v, idx_i_v, u_rows_v, i_rows_v, out_v,
               sem_u, sem_i):
    wid = lax.axis_index("s") * NC + lax.axis_index("c")
    base = wid * B_PER_W

    pltpu.sync_copy(user_idx_hbm.at[pl.ds(base, B_PER_W)], idx_u_v)
    pltpu.sync_copy(item_idx_hbm.at[pl.ds(base, B_PER_W)], idx_i_v)

    cp_u = pltpu.async_copy(user_table_hbm.at[idx_u_v], u_rows_v, sem_u)
    cp_i = pltpu.async_copy(item_table_hbm.at[idx_i_v], i_rows_v, sem_i)
    cp_u.wait()
    cp_i.wait()

    for g in range(B_PER_W // L):
        rows = lax.iota(jnp.int32, L) + (g * L)

        def body(c, acc):
            col = jnp.full((L,), c, dtype=jnp.int32)
            uv = plsc.load_gather(u_rows_v, [rows, col])
            iv = plsc.load_gather(i_rows_v, [rows, col])
            return acc + uv * iv

        dot = lax.fori_loop(0, D, body, jnp.zeros((L,), jnp.float32))
        out_v[pl.ds(g * L, L)] = 1.0 / (1.0 + jnp.exp(-dot))

    pltpu.sync_copy(out_v, out_hbm.at[pl.ds(base, B_PER_W)])


@jax.jit
def kernel(x, user_table, item_table):
    user_idx = x[:, 0].astype(jnp.int32)
    item_idx = x[:, 1].astype(jnp.int32)

    mesh = plsc.VectorSubcoreMesh(
        core_axis_name="c", subcore_axis_name="s",
        num_cores=NC, num_subcores=NS)
    run = functools.partial(
        pl.kernel,
        out_type=jax.ShapeDtypeStruct((B,), jnp.float32),
        mesh=mesh,
        compiler_params=pltpu.CompilerParams(
            needs_layout_passes=False, use_tc_tiling_on_sc=False),
        scratch_types=[
            pltpu.VMEM((B_PER_W,), jnp.int32),
            pltpu.VMEM((B_PER_W,), jnp.int32),
            pltpu.VMEM((B_PER_W, D), jnp.float32),
            pltpu.VMEM((B_PER_W, D), jnp.float32),
            pltpu.VMEM((B_PER_W,), jnp.float32),
            pltpu.SemaphoreType.DMA,
            pltpu.SemaphoreType.DMA,
        ],
    )(_mf_kernel)
    return run(user_idx, item_idx, user_table, item_table)


# speedup vs baseline: 1.5787x; 1.5787x over previous
import functools

import jax
import jax.numpy as jnp
from jax import lax
from jax.experimental import pallas as pl
from jax.experimental.pallas import tpu as pltpu
from jax.experimental.pallas import tpu_sc as plsc

NC = 2
NS = 16
L = 16
NW = NC * NS

B = 4096
D = 64
B_PER_W = B // NW


def _mf_kernel(user_idx_hbm, item_idx_hbm, user_table_hbm, item_table_hbm,
               out_hbm, idx_u_v, idx_i_v, u_rows_v, i_rows_v, out_v,
               sem_u, sem_i):
    wid = lax.axis_index("s") * NC + lax.axis_index("c")
    base = wid * B_PER_W

    pltpu.sync_copy(user_idx_hbm.at[pl.ds(base, B_PER_W)], idx_u_v)
    pltpu.sync_copy(item_idx_hbm.at[pl.ds(base, B_PER_W)], idx_i_v)

    def issue(j, carry):
        uvec = idx_u_v[pl.ds(j * L, L)]
        ivec = idx_i_v[pl.ds(j * L, L)]
        for k in range(L):
            r = j * L + k
            pltpu.async_copy(user_table_hbm.at[pl.ds(uvec[k], 1), :],
                             u_rows_v.at[pl.ds(r, 1), :], sem_u)
            pltpu.async_copy(item_table_hbm.at[pl.ds(ivec[k], 1), :],
                             i_rows_v.at[pl.ds(r, 1), :], sem_i)
        return carry

    lax.fori_loop(0, B_PER_W // L, issue, 0)

    pltpu.make_async_copy(user_table_hbm.at[pl.ds(0, B_PER_W), :],
                          u_rows_v, sem_u).wait()
    pltpu.make_async_copy(item_table_hbm.at[pl.ds(0, B_PER_W), :],
                          i_rows_v, sem_i).wait()

    for g in range(B_PER_W // L):
        rows = lax.iota(jnp.int32, L) + (g * L)

        def body(c, acc):
            col = jnp.full((L,), c, dtype=jnp.int32)
            uv = plsc.load_gather(u_rows_v, [rows, col])
            iv = plsc.load_gather(i_rows_v, [rows, col])
            return acc + uv * iv

        dot = lax.fori_loop(0, D, body, jnp.zeros((L,), jnp.float32))
        out_v[pl.ds(g * L, L)] = 1.0 / (1.0 + jnp.exp(-dot))

    pltpu.sync_copy(out_v, out_hbm.at[pl.ds(base, B_PER_W)])


@jax.jit
def kernel(x, user_table, item_table):
    user_idx = x[:, 0].astype(jnp.int32)
    item_idx = x[:, 1].astype(jnp.int32)

    mesh = plsc.VectorSubcoreMesh(
        core_axis_name="c", subcore_axis_name="s",
        num_cores=NC, num_subcores=NS)
    run = functools.partial(
        pl.kernel,
        out_type=jax.ShapeDtypeStruct((B,), jnp.float32),
        mesh=mesh,
        compiler_params=pltpu.CompilerParams(needs_layout_passes=False),
        scratch_types=[
            pltpu.VMEM((B_PER_W,), jnp.int32),
            pltpu.VMEM((B_PER_W,), jnp.int32),
            pltpu.VMEM((B_PER_W, D), jnp.float32),
            pltpu.VMEM((B_PER_W, D), jnp.float32),
            pltpu.VMEM((B_PER_W,), jnp.float32),
            pltpu.SemaphoreType.DMA,
            pltpu.SemaphoreType.DMA,
        ],
    )(_mf_kernel)
    return run(user_idx, item_idx, user_table, item_table)


# speedup vs baseline: 7.8662x; 4.9827x over previous
import functools

import jax
import jax.numpy as jnp
from jax import lax
from jax.experimental import pallas as pl
from jax.experimental.pallas import tpu as pltpu
from jax.experimental.pallas import tpu_sc as plsc

NC = 2
NS = 16
L = 16
NW = NC * NS

B = 4096
D = 64
B_PER_W = B // NW
WAVE = 4


def _mf_kernel(user_idx_hbm, item_idx_hbm, user_t_hbm, item_t_hbm,
               out_hbm, idx_u_v, idx_i_v, wu_v, wi_v, out_v, sem_u, sem_i):
    wid = lax.axis_index("s") * NC + lax.axis_index("c")
    base = wid * B_PER_W

    pltpu.sync_copy(user_idx_hbm.at[pl.ds(base, B_PER_W)], idx_u_v)
    pltpu.sync_copy(item_idx_hbm.at[pl.ds(base, B_PER_W)], idx_i_v)

    def group(g, carry):
        uvec = idx_u_v[pl.ds(g * L, L)]
        ivec = idx_i_v[pl.ds(g * L, L)]
        acc = jnp.zeros((L,), jnp.float32)
        for w in range(L // WAVE):
            descs = []
            for b in range(WAVE):
                k = w * WAVE + b
                qu = pl.multiple_of((uvec[k] // 128) * 128, 128)
                qi = pl.multiple_of((ivec[k] // 128) * 128, 128)
                descs.append(pltpu.async_copy(
                    user_t_hbm.at[:, pl.ds(qu, 128)], wu_v.at[b], sem_u))
                descs.append(pltpu.async_copy(
                    item_t_hbm.at[:, pl.ds(qi, 128)], wi_v.at[b], sem_i))
            for dsc in descs:
                dsc.wait()
            for b in range(WAVE):
                k = w * WAVE + b
                su = jnp.full((L,), uvec[k] % 128, dtype=jnp.int32)
                si = jnp.full((L,), ivec[k] % 128, dtype=jnp.int32)
                part = jnp.zeros((L,), jnp.float32)
                for c in range(D // L):
                    crange = lax.iota(jnp.int32, L) + (c * L)
                    uvals = plsc.load_gather(wu_v.at[b], [crange, su])
                    ivals = plsc.load_gather(wi_v.at[b], [crange, si])
                    part = part + uvals * ivals
                dot = jnp.sum(part)
                acc = jnp.where(lax.iota(jnp.int32, L) == k, dot, acc)
        out_v[pl.ds(g * L, L)] = 1.0 / (1.0 + jnp.exp(-acc))
        return carry

    lax.fori_loop(0, B_PER_W // L, group, 0)

    pltpu.sync_copy(out_v, out_hbm.at[pl.ds(base, B_PER_W)])


@jax.jit
def kernel(x, user_table, item_table):
    user_idx = x[:, 0].astype(jnp.int32)
    item_idx = x[:, 1].astype(jnp.int32)

    mesh = plsc.VectorSubcoreMesh(
        core_axis_name="c", subcore_axis_name="s",
        num_cores=NC, num_subcores=NS)
    run = functools.partial(
        pl.kernel,
        out_type=jax.ShapeDtypeStruct((B,), jnp.float32),
        mesh=mesh,
        compiler_params=pltpu.CompilerParams(
            needs_layout_passes=False, disable_bounds_checks=True),
        scratch_types=[
            pltpu.VMEM((B_PER_W,), jnp.int32),
            pltpu.VMEM((B_PER_W,), jnp.int32),
            pltpu.VMEM((WAVE, D, 128), jnp.float32),
            pltpu.VMEM((WAVE, D, 128), jnp.float32),
            pltpu.VMEM((B_PER_W,), jnp.float32),
            pltpu.SemaphoreType.DMA,
            pltpu.SemaphoreType.DMA,
        ],
    )(_mf_kernel)
    return run(user_idx, item_idx, user_table.T, item_table.T)


# speedup vs baseline: 8.7440x; 1.1116x over previous
import functools

import jax
import jax.numpy as jnp
from jax import lax
from jax.experimental import pallas as pl
from jax.experimental.pallas import tpu as pltpu
from jax.experimental.pallas import tpu_sc as plsc

NC = 2
NS = 16
L = 16
NW = NC * NS

B = 4096
D = 64
B_PER_W = B // NW
WAVE = 2
DEPTH = 3
NWAVES = L // WAVE


def _mf_kernel(user_idx_hbm, item_idx_hbm, user_t_hbm, item_t_hbm,
               out_hbm, idx_u_v, idx_i_v, wu_v, wi_v, out_v,
               su0, su1, su2, si0, si1, si2):
    sems_u = [su0, su1, su2]
    sems_i = [si0, si1, si2]
    wid = lax.axis_index("s") * NC + lax.axis_index("c")
    base = wid * B_PER_W

    pltpu.sync_copy(user_idx_hbm.at[pl.ds(base, B_PER_W)], idx_u_v)
    pltpu.sync_copy(item_idx_hbm.at[pl.ds(base, B_PER_W)], idx_i_v)

    def group(g, carry):
        uvec = idx_u_v[pl.ds(g * L, L)]
        ivec = idx_i_v[pl.ds(g * L, L)]
        acc = jnp.zeros((L,), jnp.float32)
        descs = {}

        def fire(w):
            p = w % DEPTH
            lst = []
            for b in range(WAVE):
                k = w * WAVE + b
                qu = pl.multiple_of((uvec[k] // 128) * 128, 128)
                qi = pl.multiple_of((ivec[k] // 128) * 128, 128)
                lst.append(pltpu.async_copy(
                    user_t_hbm.at[:, pl.ds(qu, 128)],
                    wu_v.at[p * WAVE + b], sems_u[p]))
                lst.append(pltpu.async_copy(
                    item_t_hbm.at[:, pl.ds(qi, 128)],
                    wi_v.at[p * WAVE + b], sems_i[p]))
            descs[w] = lst

        for w in range(DEPTH):
            fire(w)
        for w in range(NWAVES):
            for dsc in descs.pop(w):
                dsc.wait()
            for b in range(WAVE):
                k = w * WAVE + b
                slot = (w % DEPTH) * WAVE + b
                su = jnp.full((L,), uvec[k] % 128, dtype=jnp.int32)
                si = jnp.full((L,), ivec[k] % 128, dtype=jnp.int32)
                part = jnp.zeros((L,), jnp.float32)
                for c in range(D // L):
                    crange = lax.iota(jnp.int32, L) + (c * L)
                    uvals = plsc.load_gather(wu_v.at[slot], [crange, su])
                    ivals = plsc.load_gather(wi_v.at[slot], [crange, si])
                    part = part + uvals * ivals
                dot = jnp.sum(part)
                acc = jnp.where(lax.iota(jnp.int32, L) == k, dot, acc)
            if w + DEPTH < NWAVES:
                fire(w + DEPTH)
        out_v[pl.ds(g * L, L)] = 1.0 / (1.0 + jnp.exp(-acc))
        return carry

    lax.fori_loop(0, B_PER_W // L, group, 0)

    pltpu.sync_copy(out_v, out_hbm.at[pl.ds(base, B_PER_W)])


@jax.jit
def kernel(x, user_table, item_table):
    user_idx = x[:, 0].astype(jnp.int32)
    item_idx = x[:, 1].astype(jnp.int32)

    mesh = plsc.VectorSubcoreMesh(
        core_axis_name="c", subcore_axis_name="s",
        num_cores=NC, num_subcores=NS)
    run = functools.partial(
        pl.kernel,
        out_type=jax.ShapeDtypeStruct((B,), jnp.float32),
        mesh=mesh,
        compiler_params=pltpu.CompilerParams(
            needs_layout_passes=False, disable_bounds_checks=True),
        scratch_types=[
            pltpu.VMEM((B_PER_W,), jnp.int32),
            pltpu.VMEM((B_PER_W,), jnp.int32),
            pltpu.VMEM((WAVE * DEPTH, D, 128), jnp.float32),
            pltpu.VMEM((WAVE * DEPTH, D, 128), jnp.float32),
            pltpu.VMEM((B_PER_W,), jnp.float32),
            pltpu.SemaphoreType.DMA,
            pltpu.SemaphoreType.DMA,
            pltpu.SemaphoreType.DMA,
            pltpu.SemaphoreType.DMA,
            pltpu.SemaphoreType.DMA,
            pltpu.SemaphoreType.DMA,
        ],
    )(_mf_kernel)
    return run(user_idx, item_idx, user_table.T, item_table.T)


# speedup vs baseline: 8.7480x; 1.0005x over previous
import functools

import jax
import jax.numpy as jnp
from jax import lax
from jax.experimental import pallas as pl
from jax.experimental.pallas import tpu as pltpu
from jax.experimental.pallas import tpu_sc as plsc

NC = 2
NS = 16
L = 16
NW = NC * NS

B = 4096
D = 64
B_PER_W = B // NW
WAVE = 2
DEPTH = 3
NWAVES = L // WAVE


def _mf_kernel(user_idx_hbm, item_idx_hbm, user_t_hbm, item_t_hbm,
               out_hbm, idx_u_v, idx_i_v, wu_v, wi_v, out_v, *sems):
    sems_u = list(sems[:DEPTH])
    sems_i = list(sems[DEPTH:])
    wid = lax.axis_index("s") * NC + lax.axis_index("c")
    base = wid * B_PER_W

    pltpu.sync_copy(user_idx_hbm.at[pl.ds(base, B_PER_W)], idx_u_v)
    pltpu.sync_copy(item_idx_hbm.at[pl.ds(base, B_PER_W)], idx_i_v)

    def group(g, carry):
        uvec = idx_u_v[pl.ds(g * L, L)]
        ivec = idx_i_v[pl.ds(g * L, L)]
        acc = jnp.zeros((L,), jnp.float32)
        descs = {}

        def fire(w):
            p = w % DEPTH
            lst = []
            for b in range(WAVE):
                k = w * WAVE + b
                qu = pl.multiple_of((uvec[k] // 128) * 128, 128)
                qi = pl.multiple_of((ivec[k] // 128) * 128, 128)
                lst.append(pltpu.async_copy(
                    user_t_hbm.at[:, pl.ds(qu, 128)],
                    wu_v.at[p * WAVE + b], sems_u[p]))
                lst.append(pltpu.async_copy(
                    item_t_hbm.at[:, pl.ds(qi, 128)],
                    wi_v.at[p * WAVE + b], sems_i[p]))
            descs[w] = lst

        for w in range(DEPTH):
            fire(w)
        for w in range(NWAVES):
            for dsc in descs.pop(w):
                dsc.wait()
            for b in range(WAVE):
                k = w * WAVE + b
                slot = (w % DEPTH) * WAVE + b
                su = jnp.full((L,), uvec[k] % 128, dtype=jnp.int32)
                si = jnp.full((L,), ivec[k] % 128, dtype=jnp.int32)
                part = jnp.zeros((L,), jnp.float32)
                for c in range(D // L):
                    crange = lax.iota(jnp.int32, L) + (c * L)
                    uvals = plsc.load_gather(wu_v.at[slot], [crange, su])
                    ivals = plsc.load_gather(wi_v.at[slot], [crange, si])
                    part = part + uvals * ivals
                dot = jnp.sum(part)
                acc = jnp.where(lax.iota(jnp.int32, L) == k, dot, acc)
            if w + DEPTH < NWAVES:
                fire(w + DEPTH)
        out_v[pl.ds(g * L, L)] = 1.0 / (1.0 + jnp.exp(-acc))
        return carry

    lax.fori_loop(0, B_PER_W // L, group, 0)

    pltpu.sync_copy(out_v, out_hbm.at[pl.ds(base, B_PER_W)])


@jax.jit
def kernel(x, user_table, item_table):
    user_idx = x[:, 0].astype(jnp.int32)
    item_idx = x[:, 1].astype(jnp.int32)

    mesh = plsc.VectorSubcoreMesh(
        core_axis_name="c", subcore_axis_name="s",
        num_cores=NC, num_subcores=NS)
    run = functools.partial(
        pl.kernel,
        out_type=jax.ShapeDtypeStruct((B,), jnp.float32),
        mesh=mesh,
        compiler_params=pltpu.CompilerParams(
            needs_layout_passes=False, disable_bounds_checks=True),
        scratch_types=[
            pltpu.VMEM((B_PER_W,), jnp.int32),
            pltpu.VMEM((B_PER_W,), jnp.int32),
            pltpu.VMEM((WAVE * DEPTH, D, 128), jnp.float32),
            pltpu.VMEM((WAVE * DEPTH, D, 128), jnp.float32),
            pltpu.VMEM((B_PER_W,), jnp.float32),
        ] + [pltpu.SemaphoreType.DMA] * (2 * DEPTH),
    )(_mf_kernel)
    return run(user_idx, item_idx, user_table.T, item_table.T)


# speedup vs baseline: 9.0207x; 1.0312x over previous
import functools

import jax
import jax.numpy as jnp
from jax import lax
from jax.experimental import pallas as pl
from jax.experimental.pallas import tpu as pltpu
from jax.experimental.pallas import tpu_sc as plsc

NC = 2
NS = 16
L = 16
NW = NC * NS

B = 4096
D = 64
B_PER_W = B // NW
WAVE = 2
DEPTH = 3
NWAVES = L // WAVE


def _mf_kernel(user_idx_hbm, item_idx_hbm, user_t_hbm, item_t_hbm,
               out_hbm, idx_u_v, idx_i_v, wu_v, wi_v, out_v, *sems):
    sems_u = list(sems[:DEPTH])
    sems_i = list(sems[DEPTH:])
    wid = lax.axis_index("s") * NC + lax.axis_index("c")
    base = wid * B_PER_W

    pltpu.sync_copy(user_idx_hbm.at[pl.ds(base, B_PER_W)], idx_u_v)
    pltpu.sync_copy(item_idx_hbm.at[pl.ds(base, B_PER_W)], idx_i_v)

    total_waves = B_PER_W // WAVE
    uv, iv, descs, accs = {}, {}, {}, {}

    def getvecs(grp):
        if grp not in uv:
            uv[grp] = idx_u_v[pl.ds(grp * L, L)]
            iv[grp] = idx_i_v[pl.ds(grp * L, L)]
        return uv[grp], iv[grp]

    def fire(t):
        grp, w = t // (L // WAVE), t % (L // WAVE)
        uvec, ivec = getvecs(grp)
        p = t % DEPTH
        lst = []
        for b in range(WAVE):
            k = w * WAVE + b
            qu = pl.multiple_of((uvec[k] // 128) * 128, 128)
            qi = pl.multiple_of((ivec[k] // 128) * 128, 128)
            lst.append(pltpu.async_copy(
                user_t_hbm.at[:, pl.ds(qu, 128)],
                wu_v.at[p * WAVE + b], sems_u[p]))
            lst.append(pltpu.async_copy(
                item_t_hbm.at[:, pl.ds(qi, 128)],
                wi_v.at[p * WAVE + b], sems_i[p]))
        descs[t] = lst

    for t in range(DEPTH):
        fire(t)
    for t in range(total_waves):
        for dsc in descs.pop(t):
            dsc.wait()
        grp, w = t // (L // WAVE), t % (L // WAVE)
        uvec, ivec = uv[grp], iv[grp]
        acc = accs.get(grp)
        if acc is None:
            acc = jnp.zeros((L,), jnp.float32)
        p = t % DEPTH
        for b in range(WAVE):
            k = w * WAVE + b
            slot = p * WAVE + b
            su = jnp.full((L,), uvec[k] % 128, dtype=jnp.int32)
            si = jnp.full((L,), ivec[k] % 128, dtype=jnp.int32)
            part = jnp.zeros((L,), jnp.float32)
            for c in range(D // L):
                crange = lax.iota(jnp.int32, L) + (c * L)
                uvals = plsc.load_gather(wu_v.at[slot], [crange, su])
                ivals = plsc.load_gather(wi_v.at[slot], [crange, si])
                part = part + uvals * ivals
            dot = jnp.sum(part)
            acc = jnp.where(lax.iota(jnp.int32, L) == k, dot, acc)
        accs[grp] = acc
        if t + DEPTH < total_waves:
            fire(t + DEPTH)
        if w == (L // WAVE) - 1:
            out_v[pl.ds(grp * L, L)] = 1.0 / (1.0 + jnp.exp(-acc))

    pltpu.sync_copy(out_v, out_hbm.at[pl.ds(base, B_PER_W)])


@jax.jit
def kernel(x, user_table, item_table):
    user_idx = x[:, 0].astype(jnp.int32)
    item_idx = x[:, 1].astype(jnp.int32)

    mesh = plsc.VectorSubcoreMesh(
        core_axis_name="c", subcore_axis_name="s",
        num_cores=NC, num_subcores=NS)
    run = functools.partial(
        pl.kernel,
        out_type=jax.ShapeDtypeStruct((B,), jnp.float32),
        mesh=mesh,
        compiler_params=pltpu.CompilerParams(
            needs_layout_passes=False, disable_bounds_checks=True),
        scratch_types=[
            pltpu.VMEM((B_PER_W,), jnp.int32),
            pltpu.VMEM((B_PER_W,), jnp.int32),
            pltpu.VMEM((WAVE * DEPTH, D, 128), jnp.float32),
            pltpu.VMEM((WAVE * DEPTH, D, 128), jnp.float32),
            pltpu.VMEM((B_PER_W,), jnp.float32),
        ] + [pltpu.SemaphoreType.DMA] * (2 * DEPTH),
    )(_mf_kernel)
    return run(user_idx, item_idx, user_table.T, item_table.T)


# speedup vs baseline: 9.0784x; 1.0064x over previous
import functools

import jax
import jax.numpy as jnp
from jax import lax
from jax.experimental import pallas as pl
from jax.experimental.pallas import tpu as pltpu
from jax.experimental.pallas import tpu_sc as plsc

NC = 2
NS = 16
L = 16
NW = NC * NS

B = 4096
D = 64
B_PER_W = B // NW
WAVE = 1
DEPTH = 7
NWAVES = L // WAVE


def _mf_kernel(user_idx_hbm, item_idx_hbm, user_t_hbm, item_t_hbm,
               out_hbm, idx_u_v, idx_i_v, wu_v, wi_v, out_v, *sems):
    sems_u = list(sems[:DEPTH])
    sems_i = list(sems[DEPTH:])
    wid = lax.axis_index("s") * NC + lax.axis_index("c")
    base = wid * B_PER_W

    pltpu.sync_copy(user_idx_hbm.at[pl.ds(base, B_PER_W)], idx_u_v)
    pltpu.sync_copy(item_idx_hbm.at[pl.ds(base, B_PER_W)], idx_i_v)

    total_waves = B_PER_W // WAVE
    uv, iv, descs, accs = {}, {}, {}, {}

    def getvecs(grp):
        if grp not in uv:
            uv[grp] = idx_u_v[pl.ds(grp * L, L)]
            iv[grp] = idx_i_v[pl.ds(grp * L, L)]
        return uv[grp], iv[grp]

    def fire(t):
        grp, w = t // (L // WAVE), t % (L // WAVE)
        uvec, ivec = getvecs(grp)
        p = t % DEPTH
        lst = []
        for b in range(WAVE):
            k = w * WAVE + b
            qu = pl.multiple_of((uvec[k] // 128) * 128, 128)
            qi = pl.multiple_of((ivec[k] // 128) * 128, 128)
            lst.append(pltpu.async_copy(
                user_t_hbm.at[:, pl.ds(qu, 128)],
                wu_v.at[p * WAVE + b], sems_u[p]))
            lst.append(pltpu.async_copy(
                item_t_hbm.at[:, pl.ds(qi, 128)],
                wi_v.at[p * WAVE + b], sems_i[p]))
        descs[t] = lst

    for t in range(DEPTH):
        fire(t)
    for t in range(total_waves):
        for dsc in descs.pop(t):
            dsc.wait()
        grp, w = t // (L // WAVE), t % (L // WAVE)
        uvec, ivec = uv[grp], iv[grp]
        acc = accs.get(grp)
        if acc is None:
            acc = jnp.zeros((L,), jnp.float32)
        p = t % DEPTH
        for b in range(WAVE):
            k = w * WAVE + b
            slot = p * WAVE + b
            su = jnp.full((L,), uvec[k] % 128, dtype=jnp.int32)
            si = jnp.full((L,), ivec[k] % 128, dtype=jnp.int32)
            part = jnp.zeros((L,), jnp.float32)
            for c in range(D // L):
                crange = lax.iota(jnp.int32, L) + (c * L)
                uvals = plsc.load_gather(wu_v.at[slot], [crange, su])
                ivals = plsc.load_gather(wi_v.at[slot], [crange, si])
                part = part + uvals * ivals
            dot = jnp.sum(part)
            acc = jnp.where(lax.iota(jnp.int32, L) == k, dot, acc)
        accs[grp] = acc
        if t + DEPTH < total_waves:
            fire(t + DEPTH)
        if w == (L // WAVE) - 1:
            out_v[pl.ds(grp * L, L)] = 1.0 / (1.0 + jnp.exp(-acc))

    pltpu.sync_copy(out_v, out_hbm.at[pl.ds(base, B_PER_W)])


@jax.jit
def kernel(x, user_table, item_table):
    user_idx = x[:, 0].astype(jnp.int32)
    item_idx = x[:, 1].astype(jnp.int32)

    mesh = plsc.VectorSubcoreMesh(
        core_axis_name="c", subcore_axis_name="s",
        num_cores=NC, num_subcores=NS)
    run = functools.partial(
        pl.kernel,
        out_type=jax.ShapeDtypeStruct((B,), jnp.float32),
        mesh=mesh,
        compiler_params=pltpu.CompilerParams(
            needs_layout_passes=False, disable_bounds_checks=True),
        scratch_types=[
            pltpu.VMEM((B_PER_W,), jnp.int32),
            pltpu.VMEM((B_PER_W,), jnp.int32),
            pltpu.VMEM((WAVE * DEPTH, D, 128), jnp.float32),
            pltpu.VMEM((WAVE * DEPTH, D, 128), jnp.float32),
            pltpu.VMEM((B_PER_W,), jnp.float32),
        ] + [pltpu.SemaphoreType.DMA] * (2 * DEPTH),
    )(_mf_kernel)
    return run(user_idx, item_idx, user_table.T, item_table.T)


# speedup vs baseline: 9.1171x; 1.0043x over previous
import functools

import jax
import jax.numpy as jnp
from jax import lax
from jax.experimental import pallas as pl
from jax.experimental.pallas import tpu as pltpu
from jax.experimental.pallas import tpu_sc as plsc

NC = 2
NS = 16
L = 16
NW = NC * NS

B = 4096
D = 64
B_PER_W = B // NW
WAVE = 1
DEPTH = 7


def _mf_kernel(user_idx_hbm, item_idx_hbm, user_t_hbm, item_t_hbm,
               out_hbm, idx_u_v, idx_i_v, wu_v, wi_v, out_v, *sems):
    sems_u = list(sems[:DEPTH])
    sems_i = list(sems[DEPTH:])
    wid = lax.axis_index("s") * NC + lax.axis_index("c")
    base = wid * B_PER_W

    pltpu.sync_copy(user_idx_hbm.at[pl.ds(base, B_PER_W)], idx_u_v)
    pltpu.sync_copy(item_idx_hbm.at[pl.ds(base, B_PER_W)], idx_i_v)

    total_waves = B_PER_W // WAVE
    uv, iv, descs, accs = {}, {}, {}, {}

    def getvecs(grp):
        if grp not in uv:
            uv[grp] = idx_u_v[pl.ds(grp * L, L)]
            iv[grp] = idx_i_v[pl.ds(grp * L, L)]
        return uv[grp], iv[grp]

    def fire(t):
        grp, w = t // (L // WAVE), t % (L // WAVE)
        uvec, ivec = getvecs(grp)
        p = t % DEPTH
        lst = []
        for b in range(WAVE):
            k = w * WAVE + b
            qu = pl.multiple_of((uvec[k] // 128) * 128, 128)
            qi = pl.multiple_of((ivec[k] // 128) * 128, 128)
            lst.append(pltpu.async_copy(
                user_t_hbm.at[:, pl.ds(qu, 128)],
                wu_v.at[p * WAVE + b], sems_u[p]))
            lst.append(pltpu.async_copy(
                item_t_hbm.at[:, pl.ds(qi, 128)],
                wi_v.at[p * WAVE + b], sems_i[p]))
        descs[t] = lst

    for t in range(DEPTH):
        fire(t)
    for t in range(total_waves):
        for dsc in descs.pop(t):
            dsc.wait()
        grp, w = t // (L // WAVE), t % (L // WAVE)
        uvec, ivec = uv[grp], iv[grp]
        acc = accs.get(grp)
        if acc is None:
            acc = jnp.zeros((L,), jnp.float32)
        p = t % DEPTH
        for b in range(WAVE):
            k = w * WAVE + b
            slot = p * WAVE + b
            su = jnp.full((L,), uvec[k] % 128, dtype=jnp.int32)
            si = jnp.full((L,), ivec[k] % 128, dtype=jnp.int32)
            part = jnp.zeros((L,), jnp.float32)
            for c in range(D // L):
                crange = lax.iota(jnp.int32, L) + (c * L)
                uvals = plsc.load_gather(wu_v.at[slot], [crange, su])
                ivals = plsc.load_gather(wi_v.at[slot], [crange, si])
                part = part + uvals * ivals
            dot = jnp.sum(part)
            acc = jnp.where(lax.iota(jnp.int32, L) == k, dot, acc)
        accs[grp] = acc
        if t + DEPTH < total_waves:
            fire(t + DEPTH)
        if w == (L // WAVE) - 1:
            out_v[pl.ds(grp * L, L)] = 1.0 / (1.0 + jnp.exp(-acc))

    pltpu.sync_copy(out_v, out_hbm.at[pl.ds(base, B_PER_W)])


@jax.jit
def kernel(x, user_table, item_table):
    user_idx = x[:, 0].astype(jnp.int32)
    item_idx = x[:, 1].astype(jnp.int32)

    mesh = plsc.VectorSubcoreMesh(
        core_axis_name="c", subcore_axis_name="s",
        num_cores=NC, num_subcores=NS)
    run = functools.partial(
        pl.kernel,
        out_type=jax.ShapeDtypeStruct((B,), jnp.float32),
        mesh=mesh,
        compiler_params=pltpu.CompilerParams(
            needs_layout_passes=False, disable_bounds_checks=True),
        scratch_types=[
            pltpu.VMEM((B_PER_W,), jnp.int32),
            pltpu.VMEM((B_PER_W,), jnp.int32),
            pltpu.VMEM((WAVE * DEPTH, D, 128), jnp.float32),
            pltpu.VMEM((WAVE * DEPTH, D, 128), jnp.float32),
            pltpu.VMEM((B_PER_W,), jnp.float32),
        ] + [pltpu.SemaphoreType.DMA] * (2 * DEPTH),
    )(_mf_kernel)
    return run(user_idx, item_idx, user_table.T, item_table.T)
